# baseline (speedup 1.0000x reference)
import jax
import jax.numpy as jnp
from jax.experimental import pallas as pl
from jax.experimental.pallas import tpu as pltpu

_N = 4096
_M = 2048
_NH = 256

_BR = 128
_GR = _N // _BR
_BP = 128
_GP = _M // _BP

_BF = jnp.bfloat16
_F32 = jnp.float32


def _split(y):
    hi = y.astype(_BF)
    lo = (y - hi.astype(_F32)).astype(_BF)
    return hi, lo


def _dot(a, b):
    return jax.lax.dot(a, b, preferred_element_type=_F32)


def _dot22(a, b):
    ahi, alo = _split(a)
    bhi, blo = _split(b)
    return _dot(ahi, bhi) + _dot(ahi, blo) + _dot(alo, bhi)


def _mega_kernel(wv_ref, adj_ref, adjv_ref, adje_ref, xh_ref, xl_ref,
                 w1_ref, b1_ref,
                 w2_ref, b2_ref, wi_ref, pet_ref, bi_ref,
                 o_ref,
                 a_vmem, hh_vmem, hl_vmem, v_vmem, xwh_vmem, xwl_vmem):
    p = pl.program_id(0)
    i = pl.program_id(1)

    @pl.when(p == 0)
    def _phase0():
        @pl.when(i == 0)
        def _():
            w1_hi, w1_lo = _split(w1_ref[...])
            xw = (_dot(xh_ref[...], w1_hi) + _dot(xh_ref[...], w1_lo)
                  + _dot(xl_ref[...], w1_hi))
            xw_hi, xw_lo = _split(xw)
            xwh_vmem[...] = xw_hi
            xwl_vmem[...] = xw_lo

        cv = wv_ref[0, 0]
        ce = 2.0 - cv
        a_bf = (adj_ref[...] * (cv * adjv_ref[...] + ce * adje_ref[...])
                ).astype(_BF)
        a_vmem[i] = a_bf
        h1 = jax.nn.relu(
            _dot(a_bf, xwh_vmem[...]) + _dot(a_bf, xwl_vmem[...])
            + 2.0 * b1_ref[...])
        h1_hi, h1_lo = _split(h1)
        hh_vmem[pl.ds(i * _BR, _BR), :] = h1_hi
        hl_vmem[pl.ds(i * _BR, _BR), :] = h1_lo

    @pl.when(p == 1)
    def _phase1():
        @pl.when(i == 0)
        def _():
            w2_hi, w2_lo = _split(w2_ref[...])
            for jb in range(0, _GR, 8):
                sl = pl.ds(jb * _BR, 8 * _BR)
                h_hi = hh_vmem[sl, :]
                h_lo = hl_vmem[sl, :]
                hw = (_dot(h_hi, w2_hi) + _dot(h_hi, w2_lo)
                      + _dot(h_lo, w2_hi))
                hw_hi, hw_lo = _split(hw)
                hh_vmem[sl, :] = hw_hi
                hl_vmem[sl, :] = hw_lo

        h2 = jax.nn.relu(
            _dot(a_vmem[i], hh_vmem[...]) + _dot(a_vmem[i], hl_vmem[...])
            + 2.0 * b2_ref[...])
        v_vmem[pl.ds(i * _BR, _BR), :] = _dot22(
            h2, wi_ref[...]).astype(_BF)

    @pl.when((p == 2) & (i < _GP))
    def _phase2():
        h3 = _dot(pet_ref[...].astype(_BF), v_vmem[...]) + bi_ref[0, 0]
        o_ref[...] = jax.nn.sigmoid(h3)


def _mega(wv2d, adj, adj_v, adj_e, xh, xl, w1, b1row, w2, b2row, wi, pet,
          bi2d):
    def panel_map(p, i):
        return (jnp.where(p == 0, i, _GR - 1), 0)

    def pet_map(p, i):
        return (jnp.where(p == 2, jnp.minimum(i, _GP - 1), 0), 0)

    def out_map(p, i):
        return (jnp.where(p == 2, jnp.minimum(i, _GP - 1), 0), 0)

    zero2 = lambda p, i: (0, 0)

    return pl.pallas_call(
        _mega_kernel,
        grid=(3, _GR),
        in_specs=[
            pl.BlockSpec((1, 1), zero2),
            pl.BlockSpec((_BR, _N), panel_map),
            pl.BlockSpec((_BR, _N), panel_map),
            pl.BlockSpec((_BR, _N), panel_map),
            pl.BlockSpec((_N, _NH), zero2),
            pl.BlockSpec((_N, _NH), zero2),
            pl.BlockSpec((_NH, _NH), zero2),
            pl.BlockSpec((1, _NH), zero2),
            pl.BlockSpec((_NH, _NH), zero2),
            pl.BlockSpec((1, _NH), zero2),
            pl.BlockSpec((_NH, 1), zero2),
            pl.BlockSpec((_BP, _N), pet_map),
            pl.BlockSpec((1, 1), zero2),
        ],
        out_specs=pl.BlockSpec((_BP, 1), out_map),
        out_shape=jax.ShapeDtypeStruct((_M, 1), jnp.float32),
        scratch_shapes=[
            pltpu.VMEM((_GR, _BR, _N), _BF),
            pltpu.VMEM((_N, _NH), _BF),
            pltpu.VMEM((_N, _NH), _BF),
            pltpu.VMEM((_N, 1), _BF),
            pltpu.VMEM((_N, _NH), _BF),
            pltpu.VMEM((_N, _NH), _BF),
        ],
        compiler_params=pltpu.CompilerParams(
            vmem_limit_bytes=100 * 1024 * 1024,
        ),
    )(wv2d, adj, adj_v, adj_e, xh, xl, w1, b1row, w2, b2row, wi, pet, bi2d)


@jax.jit
def kernel(x, adj, adj_v, adj_e, PeT, wv, W1, b1, W2, b2, Wi, bi):
    wv2d = wv.reshape(1, 1).astype(jnp.float32)
    b1row = b1.reshape(1, _NH)
    b2row = b2.reshape(1, _NH)
    bi2d = bi.reshape(1, 1)
    x_hi = x.astype(_BF)
    x_lo = (x - x_hi.astype(_F32)).astype(_BF)

    return _mega(wv2d, adj, adj_v, adj_e, x_hi, x_lo, W1, b1row, W2, b2row,
                 Wi, PeT, bi2d)


# speedup vs baseline: 1.0436x; 1.0436x over previous
import jax
import jax.numpy as jnp
from jax.experimental import pallas as pl
from jax.experimental.pallas import tpu as pltpu

_N = 4096
_M = 2048
_NH = 256

_BR = 128
_GR = _N // _BR
_BP = 128
_GP = _M // _BP

_BF = jnp.bfloat16
_F32 = jnp.float32


def _split(y):
    hi = y.astype(_BF)
    lo = (y - hi.astype(_F32)).astype(_BF)
    return hi, lo


def _dot(a, b):
    return jax.lax.dot(a, b, preferred_element_type=_F32)


def _dot22(a, b):
    ahi, alo = _split(a)
    bhi, blo = _split(b)
    return _dot(ahi, bhi) + _dot(ahi, blo) + _dot(alo, bhi)


def _mega_kernel(wv_ref, adj_ref, adjv_ref, adje_ref, xh_ref, xl_ref,
                 w1_ref, b1_ref,
                 w2_ref, b2_ref, wi_ref, pet_ref, bi_ref,
                 o_ref,
                 a_vmem, hh_vmem, hl_vmem, v_vmem, xwh_vmem, xwl_vmem):
    p = pl.program_id(0)
    i = pl.program_id(1)

    @pl.when(p == 0)
    def _phase0():
        @pl.when(i == 0)
        def _():
            w1_hi, w1_lo = _split(w1_ref[...])
            xw = (_dot(xh_ref[...], w1_hi) + _dot(xh_ref[...], w1_lo)
                  + _dot(xl_ref[...], w1_hi))
            xw_hi, xw_lo = _split(xw)
            xwh_vmem[...] = xw_hi
            xwl_vmem[...] = xw_lo

        cv = wv_ref[0, 0]
        ce = 2.0 - cv
        a_bf = (adj_ref[...] * (cv * adjv_ref[...] + ce * adje_ref[...])
                ).astype(_BF)
        a_vmem[i // 2, pl.ds((i % 2) * _BR, _BR), :] = a_bf
        h1 = jax.nn.relu(
            _dot(a_bf, xwh_vmem[...]) + _dot(a_bf, xwl_vmem[...])
            + 2.0 * b1_ref[...])
        h1_hi, h1_lo = _split(h1)
        hh_vmem[pl.ds(i * _BR, _BR), :] = h1_hi
        hl_vmem[pl.ds(i * _BR, _BR), :] = h1_lo

    @pl.when(p == 1)
    def _phase1():
        @pl.when(i == 0)
        def _():
            w2_hi, w2_lo = _split(w2_ref[...])
            for jb in range(0, _GR, 8):
                sl = pl.ds(jb * _BR, 8 * _BR)
                h_hi = hh_vmem[sl, :]
                h_lo = hl_vmem[sl, :]
                hw = (_dot(h_hi, w2_hi) + _dot(h_hi, w2_lo)
                      + _dot(h_lo, w2_hi))
                hw_hi, hw_lo = _split(hw)
                hh_vmem[sl, :] = hw_hi
                hl_vmem[sl, :] = hw_lo

        @pl.when(i < _GR // 2)
        def _():
            h2 = jax.nn.relu(
                _dot(a_vmem[i], hh_vmem[...])
                + _dot(a_vmem[i], hl_vmem[...])
                + 2.0 * b2_ref[...])
            v_vmem[pl.ds(i * 2 * _BR, 2 * _BR), :] = _dot22(
                h2, wi_ref[...]).astype(_BF)

    @pl.when((p == 2) & (i < _GP))
    def _phase2():
        h3 = _dot(pet_ref[...].astype(_BF), v_vmem[...]) + bi_ref[0, 0]
        o_ref[...] = jax.nn.sigmoid(h3)


def _mega(wv2d, adj, adj_v, adj_e, xh, xl, w1, b1row, w2, b2row, wi, pet,
          bi2d):
    def panel_map(p, i):
        return (jnp.where(p == 0, i, _GR - 1), 0)

    def pet_map(p, i):
        return (jnp.where(p == 2, jnp.minimum(i, _GP - 1), 0), 0)

    def out_map(p, i):
        return (jnp.where(p == 2, jnp.minimum(i, _GP - 1), 0), 0)

    zero2 = lambda p, i: (0, 0)

    return pl.pallas_call(
        _mega_kernel,
        grid=(3, _GR),
        in_specs=[
            pl.BlockSpec((1, 1), zero2),
            pl.BlockSpec((_BR, _N), panel_map),
            pl.BlockSpec((_BR, _N), panel_map),
            pl.BlockSpec((_BR, _N), panel_map),
            pl.BlockSpec((_N, _NH), zero2),
            pl.BlockSpec((_N, _NH), zero2),
            pl.BlockSpec((_NH, _NH), zero2),
            pl.BlockSpec((1, _NH), zero2),
            pl.BlockSpec((_NH, _NH), zero2),
            pl.BlockSpec((1, _NH), zero2),
            pl.BlockSpec((_NH, 1), zero2),
            pl.BlockSpec((_BP, _N), pet_map),
            pl.BlockSpec((1, 1), zero2),
        ],
        out_specs=pl.BlockSpec((_BP, 1), out_map),
        out_shape=jax.ShapeDtypeStruct((_M, 1), jnp.float32),
        scratch_shapes=[
            pltpu.VMEM((_GR // 2, 2 * _BR, _N), _BF),
            pltpu.VMEM((_N, _NH), _BF),
            pltpu.VMEM((_N, _NH), _BF),
            pltpu.VMEM((_N, 1), _BF),
            pltpu.VMEM((_N, _NH), _BF),
            pltpu.VMEM((_N, _NH), _BF),
        ],
        compiler_params=pltpu.CompilerParams(
            vmem_limit_bytes=100 * 1024 * 1024,
        ),
    )(wv2d, adj, adj_v, adj_e, xh, xl, w1, b1row, w2, b2row, wi, pet, bi2d)


@jax.jit
def kernel(x, adj, adj_v, adj_e, PeT, wv, W1, b1, W2, b2, Wi, bi):
    wv2d = wv.reshape(1, 1).astype(jnp.float32)
    b1row = b1.reshape(1, _NH)
    b2row = b2.reshape(1, _NH)
    bi2d = bi.reshape(1, 1)
    x_hi = x.astype(_BF)
    x_lo = (x - x_hi.astype(_F32)).astype(_BF)

    return _mega(wv2d, adj, adj_v, adj_e, x_hi, x_lo, W1, b1row, W2, b2row,
                 Wi, PeT, bi2d)
